# baseline (speedup 1.0000x reference)
import functools

import jax
import jax.numpy as jnp
from jax import lax
from jax.experimental import pallas as pl
from jax.experimental.pallas import tpu as pltpu
from jax.experimental.pallas import tpu_sc as plsc

_B, _T, _D, _E, _C = 4096, 16, 1024, 8, 256
_BS = 256
_B_SC = 512
_B_TC = _B - _B_SC

_NC, _NS, _L = 2, 16, 16
_NW = _NC * _NS


def _masked_heads(xp, sid, w_ref, b_ref):
    acc = jnp.zeros((xp.shape[0], _C), jnp.float32)
    for e in range(_E):
        mask = (sid == e).astype(jnp.float32)[:, None]
        y = jax.lax.dot_general(
            xp, w_ref[e],
            dimension_numbers=(((1,), (1,)), ((), ())),
            preferred_element_type=jnp.float32,
        )
        acc = acc + mask * (y + b_ref[e][None, :])
    return acc


def _fused_body(sid_ref, x_ref, w_ref, b_ref, o_ref):
    xp = jnp.mean(x_ref[...], axis=1)
    o_ref[...] = _masked_heads(xp, sid_ref[0, 0, :], w_ref, b_ref)


def _mm_body(sid_ref, xp_ref, w_ref, b_ref, prev_ref, o_ref):
    del prev_ref
    o_ref[...] = _masked_heads(xp_ref[...], sid_ref[0, 0, :], w_ref, b_ref)


def _tc_fused(x, sid3, W, b):
    nb = _B_TC // _BS
    return pl.pallas_call(
        _fused_body,
        grid=(nb,),
        in_specs=[
            pl.BlockSpec((1, 1, _BS), lambda i: (i, 0, 0)),
            pl.BlockSpec((_BS, _T, _D), lambda i: (i, 0, 0)),
            pl.BlockSpec((_E, _C, _D), lambda i: (0, 0, 0)),
            pl.BlockSpec((_E, _C), lambda i: (0, 0)),
        ],
        out_specs=pl.BlockSpec((_BS, _C), lambda i: (i, 0)),
        out_shape=jax.ShapeDtypeStruct((_B, _C), jnp.float32),
    )(sid3, x, W, b)


def _tc_matmul_tail(xp_sc, sid3, W, b, prev_out):
    blk_off = _B_TC // _B_SC
    return pl.pallas_call(
        _mm_body,
        grid=(1,),
        in_specs=[
            pl.BlockSpec((1, 1, _B_SC), lambda i: (i + blk_off, 0, 0)),
            pl.BlockSpec((_B_SC, _D), lambda i: (i, 0)),
            pl.BlockSpec((_E, _C, _D), lambda i: (0, 0, 0)),
            pl.BlockSpec((_E, _C), lambda i: (0, 0)),
            pl.BlockSpec(memory_space=pl.ANY),
        ],
        out_specs=pl.BlockSpec((_B_SC, _C), lambda i: (i + blk_off, 0)),
        out_shape=jax.ShapeDtypeStruct((_B, _C), jnp.float32),
        input_output_aliases={4: 0},
    )(sid3, xp_sc, W, b, prev_out)


def _make_sc_pool(b_sc, row_off):
    rows_w = b_sc // _NW
    mesh = plsc.VectorSubcoreMesh(core_axis_name="c", subcore_axis_name="s")

    @functools.partial(
        pl.kernel, mesh=mesh,
        out_type=jax.ShapeDtypeStruct((b_sc, _D), jnp.float32),
        scratch_types=[
            pltpu.VMEM((_T, _D), jnp.float32),
            pltpu.VMEM((_T, _D), jnp.float32),
            pltpu.VMEM((_T, _D), jnp.float32),
            pltpu.VMEM((_T, _D), jnp.float32),
            pltpu.VMEM((rows_w, _D), jnp.float32),
            pltpu.SemaphoreType.DMA,
            pltpu.SemaphoreType.DMA,
            pltpu.SemaphoreType.DMA,
            pltpu.SemaphoreType.DMA,
        ],
    )
    def sc_pool(x_hbm, o_hbm, b0, b1, b2, b3, ov, s0, s1, s2, s3):
        bufs = (b0, b1, b2, b3)
        sems = (s0, s1, s2, s3)
        wid = lax.axis_index("s") * _NC + lax.axis_index("c")
        base = wid * rows_w
        last = rows_w - 1

        for j in range(3):
            pltpu.async_copy(x_hbm.at[row_off + base + j], bufs[j], sems[j])

        def pool_one(buf, r_local):
            def chunk(c, carry):
                o = pl.ds(c * _L, _L)
                acc = buf[0, o]
                for t in range(1, _T):
                    acc = acc + buf[t, o]
                ov[r_local, o] = acc * (1.0 / _T)
                return carry
            lax.fori_loop(0, _D // _L, chunk, 0, unroll=8)

        def step(k, carry):
            for j in range(4):
                r = 4 * k + j
                pltpu.make_async_copy(
                    x_hbm.at[row_off + base + r], bufs[j], sems[j]).wait()
                pool_one(bufs[j], r)
                nxt = jnp.minimum(r + 3, last)
                pltpu.async_copy(x_hbm.at[row_off + base + nxt],
                                 bufs[(j + 3) % 4], sems[(j + 3) % 4])
            return carry

        lax.fori_loop(0, rows_w // 4, step, 0)
        for j in range(3):
            pltpu.make_async_copy(
                x_hbm.at[row_off + base + last], bufs[j], sems[j]).wait()
        pltpu.sync_copy(ov, o_hbm.at[pl.ds(base, rows_w)])

    return sc_pool


_sc_pool = _make_sc_pool(_B_SC, _B_TC)


@jax.jit
def kernel(x, system_id, W, b):
    sid = system_id.astype(jnp.int32)

    xp_sc = _sc_pool(x)

    sid3a = sid.reshape(_B // _BS, 1, _BS)
    out1 = _tc_fused(x, sid3a, W, b)

    sid3b = sid.reshape(_B // _B_SC, 1, _B_SC)
    return _tc_matmul_tail(xp_sc, sid3b, W, b, out1)


# speedup vs baseline: 1.2924x; 1.2924x over previous
import jax
import jax.numpy as jnp
from jax.experimental import pallas as pl

_B, _T, _D, _E, _C = 4096, 16, 1024, 8, 256
_BS = 256


def _fused_body(sid_ref, x_ref, w_ref, b_ref, o_ref):
    xp = jnp.mean(x_ref[...], axis=1)
    sid = sid_ref[0, 0, :]
    acc = jnp.zeros((_BS, _C), jnp.float32)
    for e in range(_E):
        mask = (sid == e).astype(jnp.float32)[:, None]
        y = jax.lax.dot_general(
            xp, w_ref[e],
            dimension_numbers=(((1,), (1,)), ((), ())),
            preferred_element_type=jnp.float32,
        )
        acc = acc + mask * (y + b_ref[e][None, :])
    o_ref[...] = acc


@jax.jit
def kernel(x, system_id, W, b):
    nb = _B // _BS
    sid3 = system_id.astype(jnp.int32).reshape(nb, 1, _BS)
    return pl.pallas_call(
        _fused_body,
        grid=(nb,),
        in_specs=[
            pl.BlockSpec((1, 1, _BS), lambda i: (i, 0, 0)),
            pl.BlockSpec((_BS, _T, _D), lambda i: (i, 0, 0)),
            pl.BlockSpec((_E, _C, _D), lambda i: (0, 0, 0)),
            pl.BlockSpec((_E, _C), lambda i: (0, 0)),
        ],
        out_specs=pl.BlockSpec((_BS, _C), lambda i: (i, 0)),
        out_shape=jax.ShapeDtypeStruct((_B, _C), jnp.float32),
    )(sid3, x, W, b)
